# Initial kernel scaffold; baseline (speedup 1.0000x reference)
import jax
import jax.numpy as jnp
from jax.experimental import pallas as pl


def kernel(x, edge_index, edge_weight, W1, b1, W2, b2):
    raise NotImplementedError("write your pallas kernel here")


# speedup vs baseline: 2.7332x; 2.7332x over previous
import jax
import jax.numpy as jnp
from jax.experimental import pallas as pl


def _bias_add_kernel(h_ref, b_ref, o_ref):
    o_ref[...] = h_ref[...] + b_ref[...]


def _gcn(x, src, dst, ew, W, b, dis):
    h = x @ W
    hp = dis[:, None] * h
    msg = hp[src] * ew[:, None]
    s = jnp.zeros_like(hp).at[dst].add(msg)
    out = dis[:, None] * (s + hp)
    return out + b


def kernel(x, edge_index, edge_weight, W1, b1, W2, b2):
    n = x.shape[0]
    src = edge_index[0]
    dst = edge_index[1]
    deg = jnp.zeros((n,), jnp.float32).at[dst].add(edge_weight) + 1.0
    dis = deg ** -0.5
    h = _gcn(x, src, dst, edge_weight, W1, b1, dis)
    h = jax.nn.relu(h)
    h2 = x_out = _gcn(h, src, dst, edge_weight, W2, jnp.zeros_like(b2), dis)
    out = pl.pallas_call(
        _bias_add_kernel,
        out_shape=jax.ShapeDtypeStruct(h2.shape, h2.dtype),
    )(h2, jnp.broadcast_to(b2, h2.shape))
    return out


# speedup vs baseline: 23.8061x; 8.7100x over previous
import functools

import jax
import jax.numpy as jnp
from jax import lax
from jax.experimental import pallas as pl
from jax.experimental.pallas import tpu as pltpu
from jax.experimental.pallas import tpu_sc as plsc

N = 10000
NP = 10240
E = 320000
EP = 327680
D = 128
C = 16

NCORES = 2
NSUB = 16
NW = NCORES * NSUB
CHUNK = 128
EDGES_PER_W = EP // NW
CHUNKS_PER_W = EDGES_PER_W // CHUNK
ROWS_PER_TILE = NP // NSUB

_MESH = plsc.VectorSubcoreMesh(core_axis_name="c", subcore_axis_name="s")


@functools.partial(
    pl.kernel,
    mesh=_MESH,
    out_type=jax.ShapeDtypeStruct((NCORES, NP), jnp.float32),
    scratch_types=[
        pltpu.VMEM((CHUNKS_PER_W, CHUNK), jnp.int32),
        pltpu.VMEM((CHUNKS_PER_W, CHUNK), jnp.float32),
        pltpu.VMEM((CHUNK,), jnp.float32),
        pltpu.VMEM_SHARED((NP,), jnp.float32),
    ],
)
def _deg_kernel(dst_hbm, ew_hbm, out_hbm, dst_v, ew_v, buf_v, acc_sh):
    c = lax.axis_index("c")
    s = lax.axis_index("s")
    w = c * NSUB + s
    zero = jnp.zeros((16,), jnp.float32)

    def zbuf(i, carry):
        buf_v[pl.ds(i * 16, 16)] = zero
        return carry
    lax.fori_loop(0, CHUNK // 16, zbuf, 0)

    def zacc(k, carry):
        pltpu.sync_copy(buf_v, acc_sh.at[pl.ds(s * ROWS_PER_TILE + k * CHUNK, CHUNK)])
        return carry
    lax.fori_loop(0, ROWS_PER_TILE // CHUNK, zacc, 0)
    plsc.subcore_barrier()

    pltpu.sync_copy(dst_hbm.at[pl.ds(w * CHUNKS_PER_W, CHUNKS_PER_W)], dst_v)
    pltpu.sync_copy(ew_hbm.at[pl.ds(w * CHUNKS_PER_W, CHUNKS_PER_W)], ew_v)

    def body(j, carry):
        pltpu.sync_copy(ew_v.at[j], acc_sh.at[dst_v.at[j]], add=True)
        return carry
    lax.fori_loop(0, CHUNKS_PER_W, body, 0)
    plsc.subcore_barrier()

    def drain(k, carry):
        sl = pl.ds(s * ROWS_PER_TILE + k * CHUNK, CHUNK)
        pltpu.sync_copy(acc_sh.at[sl], buf_v)
        pltpu.sync_copy(buf_v, out_hbm.at[c, sl])
        return carry
    lax.fori_loop(0, ROWS_PER_TILE // CHUNK, drain, 0)


def _make_scatter(F):
    nv = F // 16

    @functools.partial(
        pl.kernel,
        mesh=_MESH,
        compiler_params=pltpu.CompilerParams(
            use_tc_tiling_on_sc=(F % 128 == 0)),
        out_type=jax.ShapeDtypeStruct((NCORES, NP, F), jnp.float32),
        scratch_types=[
            pltpu.VMEM((CHUNKS_PER_W, CHUNK), jnp.int32),
            pltpu.VMEM((CHUNKS_PER_W, CHUNK), jnp.int32),
            pltpu.VMEM((EDGES_PER_W,), jnp.float32),
            pltpu.VMEM((CHUNK, F), jnp.float32),
            pltpu.VMEM_SHARED((NP, F), jnp.float32),
        ],
    )
    def _scatter(src_hbm, dst_hbm, ew_hbm, table_hbm, out_hbm,
                 src_v, dst_v, ew_v, rows_v, acc_sh):
        c = lax.axis_index("c")
        s = lax.axis_index("s")
        w = c * NSUB + s
        zero = jnp.zeros((16,), jnp.float32)

        def zrow(i, carry):
            rows_v[i // nv, pl.ds((i % nv) * 16, 16)] = zero
            return carry
        lax.fori_loop(0, CHUNK * nv, zrow, 0)

        def zacc(k, carry):
            pltpu.sync_copy(
                rows_v, acc_sh.at[pl.ds(s * ROWS_PER_TILE + k * CHUNK, CHUNK)])
            return carry
        lax.fori_loop(0, ROWS_PER_TILE // CHUNK, zacc, 0)
        plsc.subcore_barrier()

        pltpu.sync_copy(src_hbm.at[pl.ds(w * CHUNKS_PER_W, CHUNKS_PER_W)], src_v)
        pltpu.sync_copy(dst_hbm.at[pl.ds(w * CHUNKS_PER_W, CHUNKS_PER_W)], dst_v)
        pltpu.sync_copy(ew_hbm.at[pl.ds(w * EDGES_PER_W, EDGES_PER_W)], ew_v)

        idxc = [jnp.full((16,), u, jnp.int32) for u in range(16)]

        def chunk_body(j, carry):
            pltpu.sync_copy(table_hbm.at[src_v.at[j]], rows_v)
            base = j * CHUNK

            def edge_group(g, inner):
                wvec = ew_v[pl.ds(base + g * 16, 16)]
                for u in range(16):
                    splat = wvec.at[idxc[u]].get(mode="promise_in_bounds")
                    r = g * 16 + u
                    for f in range(nv):
                        sl = pl.ds(f * 16, 16)
                        rows_v[r, sl] = rows_v[r, sl] * splat
                return inner
            lax.fori_loop(0, CHUNK // 16, edge_group, 0)
            pltpu.sync_copy(rows_v, acc_sh.at[dst_v.at[j]], add=True)
            return carry
        lax.fori_loop(0, CHUNKS_PER_W, chunk_body, 0)
        plsc.subcore_barrier()

        def drain(k, carry):
            sl = pl.ds(s * ROWS_PER_TILE + k * CHUNK, CHUNK)
            pltpu.sync_copy(acc_sh.at[sl], rows_v)
            pltpu.sync_copy(rows_v, out_hbm.at[c, sl])
            return carry
        lax.fori_loop(0, ROWS_PER_TILE // CHUNK, drain, 0)

    return _scatter


_scatter_l1 = _make_scatter(D)
_scatter_l2 = _make_scatter(C)


_BLK = 1280
_GRID = NP // _BLK


def _dis(degp_ref):
    return lax.rsqrt(degp_ref[0, :] + degp_ref[1, :] + 1.0)


def _prep1_body(x_ref, w_ref, degp_ref, o_ref):
    h = jnp.dot(x_ref[...], w_ref[...], preferred_element_type=jnp.float32)
    o_ref[...] = h * _dis(degp_ref)[:, None]


def _mid_body(s1_ref, h1p_ref, degp_ref, b1_ref, w2_ref, o_ref):
    dis = _dis(degp_ref)
    tot = s1_ref[0] + s1_ref[1] + h1p_ref[...]
    z = jnp.maximum(tot * dis[:, None] + b1_ref[...], 0.0)
    h2 = jnp.dot(z, w2_ref[...], preferred_element_type=jnp.float32)
    o_ref[...] = h2 * dis[:, None]


def _final_body(s2_ref, h2p_ref, degp_ref, b2_ref, o_ref):
    dis = _dis(degp_ref)
    o_ref[...] = ((s2_ref[0] + s2_ref[1] + h2p_ref[...]) * dis[:, None]
                  + b2_ref[...])


_prep1 = pl.pallas_call(
    _prep1_body,
    grid=(_GRID,),
    in_specs=[
        pl.BlockSpec((_BLK, D), lambda i: (i, 0)),
        pl.BlockSpec((D, D), lambda i: (0, 0)),
        pl.BlockSpec((NCORES, _BLK), lambda i: (0, i)),
    ],
    out_specs=pl.BlockSpec((_BLK, D), lambda i: (i, 0)),
    out_shape=jax.ShapeDtypeStruct((NP, D), jnp.float32),
)

_mid = pl.pallas_call(
    _mid_body,
    grid=(_GRID,),
    in_specs=[
        pl.BlockSpec((NCORES, _BLK, D), lambda i: (0, i, 0)),
        pl.BlockSpec((_BLK, D), lambda i: (i, 0)),
        pl.BlockSpec((NCORES, _BLK), lambda i: (0, i)),
        pl.BlockSpec((1, D), lambda i: (0, 0)),
        pl.BlockSpec((D, C), lambda i: (0, 0)),
    ],
    out_specs=pl.BlockSpec((_BLK, C), lambda i: (i, 0)),
    out_shape=jax.ShapeDtypeStruct((NP, C), jnp.float32),
)

_final = pl.pallas_call(
    _final_body,
    grid=(_GRID,),
    in_specs=[
        pl.BlockSpec((NCORES, _BLK, C), lambda i: (0, i, 0)),
        pl.BlockSpec((_BLK, C), lambda i: (i, 0)),
        pl.BlockSpec((NCORES, _BLK), lambda i: (0, i)),
        pl.BlockSpec((1, C), lambda i: (0, 0)),
    ],
    out_specs=pl.BlockSpec((_BLK, C), lambda i: (i, 0)),
    out_shape=jax.ShapeDtypeStruct((NP, C), jnp.float32),
)


def kernel(x, edge_index, edge_weight, W1, b1, W2, b2):
    f32 = jnp.float32
    src = edge_index[0]
    dst = edge_index[1]
    padn = EP - E
    pad_idx = (N + (jnp.arange(padn, dtype=jnp.int32) % (NP - N))).astype(jnp.int32)
    src_p = jnp.concatenate([src, pad_idx]).reshape(EP // CHUNK, CHUNK)
    dst_p = jnp.concatenate([dst, pad_idx]).reshape(EP // CHUNK, CHUNK)
    ew_flat = jnp.concatenate([edge_weight, jnp.zeros((padn,), f32)])
    ew2 = ew_flat.reshape(EP // CHUNK, CHUNK)
    x_p = jnp.pad(x, ((0, NP - N), (0, 0)))

    degp = _deg_kernel(dst_p, ew2)
    h1p = _prep1(x_p, W1, degp)
    s1p = _scatter_l1(src_p, dst_p, ew_flat, h1p)
    h2p = _mid(s1p, h1p, degp, b1.reshape(1, D), W2)
    s2p = _scatter_l2(src_p, dst_p, ew_flat, h2p)
    outp = _final(s2p, h2p, degp, b2.reshape(1, C))
    return outp[:N]
